# baseline (speedup 1.0000x reference)
import functools

import jax
import jax.numpy as jnp
from jax import lax
from jax.experimental import pallas as pl
from jax.experimental.pallas import tpu as pltpu
from jax.experimental.pallas import tpu_sc as plsc

N_NODES = 10000
N_GRAPHS = 256
N_EDGES = 160000
EPS = 1e-5

NC, NS, LANES = 2, 16, 16
NW = NC * NS
BN_ = 160
NCHUNK = 64
NPAD = NCHUNK * BN_
EB = N_EDGES // NW
K = 16
SE = 2048
LE = N_EDGES + SE + 64

_MESH = plsc.VectorSubcoreMesh(
    core_axis_name="c", subcore_axis_name="s", num_cores=NC, num_subcores=NS)
_SC_PARAMS = pltpu.CompilerParams(needs_layout_passes=False)


def _i32(v):
    return jnp.int32(v)


_GDN = lax.GatherDimensionNumbers(
    offset_dims=(), collapsed_slice_dims=(0,), start_index_map=(0,))


def _dyn_gather(x, idx):
    return lax.gather(x, idx[:, None], _GDN, (1,),
                      mode=lax.GatherScatterMode.PROMISE_IN_BOUNDS)


def _wid():
    return lax.axis_index("s") * _i32(NC) + lax.axis_index("c")


def _make_agg(W, BN, NCH, BPAD):
    @functools.partial(
        pl.kernel,
        out_type=jax.ShapeDtypeStruct((NPAD, W), jnp.float32),
        mesh=_MESH,
        scratch_types=[
            pltpu.VMEM((BPAD,), jnp.int32),
            pltpu.VMEM((SE,), jnp.int32),
            pltpu.VMEM((SE,), jnp.int32),
            pltpu.VMEM((SE,), jnp.float32),
            pltpu.VMEM((2, K, W), jnp.float32),
            pltpu.VMEM((BN, W), jnp.float32),
            pltpu.SemaphoreType.DMA,
            pltpu.SemaphoreType.DMA,
        ],
        compiler_params=_SC_PARAMS,
    )
    def agg(hs_hbm, row_hbm, col_hbm, nrm_hbm, bnd_hbm, out_hbm,
            bnd_v, stg_r, stg_c, stg_n, rows2, acc_v, sem0, sem1):
        wid = _wid()
        sems = (sem0, sem1)
        pltpu.sync_copy(bnd_hbm, bnd_v)
        nbmax = SE // K

        def chunk(ci, carry):
            c = wid + ci * _i32(NW)
            n0 = c * _i32(BN)

            def zero(i, cz):
                for f in range(W // LANES):
                    acc_v[i, pl.ds(f * LANES, LANES)] = jnp.zeros(
                        (LANES,), jnp.float32)
                return cz

            lax.fori_loop(_i32(0), _i32(BN), zero, _i32(0))

            bv = bnd_v[pl.ds(c, LANES)]
            e0 = bv[0]
            e1 = bv[1]
            e0a = (e0 // _i32(8)) * _i32(8)
            nstage = (e1 - e0a + _i32(SE - 1)) // _i32(SE)

            def start(b, par):
                pltpu.async_copy(
                    hs_hbm.at[stg_r.at[pl.ds(b * _i32(K), K)]],
                    rows2.at[_i32(par)], sems[par])

            def wait(par):
                pltpu.make_async_copy(
                    hs_hbm.at[pl.ds(_i32(0), K)], rows2.at[_i32(par)],
                    sems[par]).wait()

            def stage(sidx, cs):
                sbase = e0a + sidx * _i32(SE)
                pltpu.sync_copy(row_hbm.at[pl.ds(sbase, SE)], stg_r)
                pltpu.sync_copy(col_hbm.at[pl.ds(sbase, SE)], stg_c)
                pltpu.sync_copy(nrm_hbm.at[pl.ds(sbase, SE)], stg_n)
                nb = jnp.minimum(_i32(nbmax),
                                 (e1 - sbase + _i32(K - 1)) // _i32(K))

                @pl.when(nb > _i32(0))
                def _():
                    start(_i32(0), 0)

                def accum(b, par):
                    cv = stg_c[pl.ds(b * _i32(K), K)] - n0
                    nv = stg_n[pl.ds(b * _i32(K), K)]
                    ev = (jnp.arange(LANES, dtype=jnp.int32) + sbase
                          + b * _i32(K))

                    def edge(j, ce):
                        jv = jnp.full((LANES,), j, jnp.int32)
                        rv = _dyn_gather(cv, jv)
                        wv = _dyn_gather(nv, jv)
                        ejv = _dyn_gather(ev, jv)
                        mv = jnp.logical_and(ejv >= e0, ejv < e1)
                        for f in range(W // LANES):
                            plsc.addupdate_scatter(
                                acc_v,
                                [rv, jnp.arange(LANES, dtype=jnp.int32)
                                 + _i32(f * LANES)],
                                rows2[_i32(par), j, pl.ds(f * LANES, LANES)]
                                * wv,
                                mask=mv)
                        return ce

                    lax.fori_loop(_i32(0), _i32(K), edge, _i32(0))

                def pair(pi, cp):
                    for par in (0, 1):
                        b = _i32(2) * pi + _i32(par)

                        @pl.when(b < nb)
                        def _():
                            @pl.when(b + _i32(1) < nb)
                            def _():
                                start(b + _i32(1), 1 - par)

                            wait(par)
                            accum(b, par)
                    return cp

                lax.fori_loop(_i32(0), (nb + _i32(1)) // _i32(2), pair,
                              _i32(0))
                return cs

            lax.fori_loop(_i32(0), nstage, stage, _i32(0))
            pltpu.sync_copy(acc_v, out_hbm.at[pl.ds(n0, BN)])
            return carry

        lax.fori_loop(_i32(0), _i32(NCH // NW), chunk, _i32(0))

    return agg


_AGG = {1024: _make_agg(1024, 64, 160, 176),
        512: _make_agg(512, 160, 64, 80),
        256: _make_agg(256, 160, 64, 80)}


def _pad_rows(a):
    return jnp.pad(a, ((0, NPAD - a.shape[0]), (0, 0)))


def kernel(x, edge_index, batch, params):
    p = params
    row = edge_index[0].astype(jnp.int32)
    col = edge_index[1].astype(jnp.int32)
    order = jnp.argsort(col, stable=True)
    row_s = row[order]
    col_s = col[order]
    row_p = jnp.pad(row_s, (0, LE - N_EDGES))
    col_p = jnp.pad(col_s, (0, LE - N_EDGES))
    b64 = jnp.searchsorted(
        col_s, (jnp.arange(161) * 64).astype(jnp.int32)).astype(jnp.int32)
    b64 = jnp.pad(b64, (0, 176 - 161))
    b160 = jnp.searchsorted(
        col_s, (jnp.arange(65) * 160).astype(jnp.int32)).astype(jnp.int32)
    b160 = jnp.pad(b160, (0, 80 - 65))
    BNDS = {1024: b64, 512: b160, 256: b160}

    order_r = jnp.argsort(row, stable=True)
    row_rs = row[order_r]
    row_rs_p = jnp.pad(row_rs, (0, LE - N_EDGES))
    rb160 = jnp.searchsorted(
        row_rs, (jnp.arange(65) * 160).astype(jnp.int32)).astype(jnp.int32)
    rb160 = jnp.pad(rb160, (0, 80 - 65))
    ones_h = jnp.ones((NPAD, 256), jnp.float32)
    ones_e = jnp.ones((LE,), jnp.float32)
    deg = 1.0 + _AGG[256](ones_h, row_rs_p, row_rs_p, ones_e, rb160)[:N_NODES, 0]
    dinv = jnp.where(deg > 0, deg ** -0.5, 0.0)

    norm_s = dinv[row_s] * dinv[col_s]
    norm_p = jnp.pad(norm_s, (0, LE - N_EDGES))
    selfw = (dinv * dinv)[:, None]

    def gcn(h, Wm, b, bng, bnb):
        z = h @ Wm
        F = z.shape[1]
        E = _AGG[F](_pad_rows(z), row_p, col_p, norm_p, BNDS[F])[:N_NODES]
        y = jax.nn.relu(E + selfw * z + b)
        m = y.mean(axis=0)
        v = y.var(axis=0)
        return bng * (y - m) / jnp.sqrt(v + EPS) + bnb

    g = gcn(x, p['W1'], p['b1'], p['bn1_g'], p['bn1_b'])
    g = gcn(g, p['W2'], p['b2'], p['bn2_g'], p['bn2_b'])
    g = gcn(g, p['W3'], p['b3'], p['bn3_g'], p['bn3_b'])
    g = gcn(g, p['W4'], p['b4'], p['bn4_g'], p['bn4_b'])
    g5 = gcn(g, p['W5'], p['b5'], p['bn5_g'], p['bn5_b'])

    gate = g5 @ p['gate_W'] + p['gate_b']
    gmax = jax.ops.segment_max(gate, batch, num_segments=N_GRAPHS)
    gmax = jnp.where(jnp.isfinite(gmax), gmax, 0.0)
    e = jnp.exp(gate - gmax[batch])
    denom = jax.ops.segment_sum(e, batch, num_segments=N_GRAPHS)
    d = denom[batch]
    alpha = e / jnp.where(d > 0, d, 1.0)
    h = jax.ops.segment_sum(alpha * g5, batch, num_segments=N_GRAPHS)
    h = jax.nn.relu(h @ p['fc2_W'] + p['fc2_b'])
    h = jax.nn.relu(h @ p['fc3_W'] + p['fc3_b'])
    return h @ p['fc4_W'] + p['fc4_b']


# speedup vs baseline: 1.0154x; 1.0154x over previous
import functools

import jax
import jax.numpy as jnp
from jax import lax
from jax.experimental import pallas as pl
from jax.experimental.pallas import tpu as pltpu
from jax.experimental.pallas import tpu_sc as plsc

N_NODES = 10000
N_GRAPHS = 256
N_EDGES = 160000
EPS = 1e-5

NC, NS, LANES = 2, 16, 16
NW = NC * NS
BN_ = 160
NCHUNK = 64
NPAD = NCHUNK * BN_
EB = N_EDGES // NW
K = 16
SE = 2048
LE = N_EDGES + SE + 64

_MESH = plsc.VectorSubcoreMesh(
    core_axis_name="c", subcore_axis_name="s", num_cores=NC, num_subcores=NS)
_SC_PARAMS = pltpu.CompilerParams(needs_layout_passes=False)


def _i32(v):
    return jnp.int32(v)


_GDN = lax.GatherDimensionNumbers(
    offset_dims=(), collapsed_slice_dims=(0,), start_index_map=(0,))


def _dyn_gather(x, idx):
    return lax.gather(x, idx[:, None], _GDN, (1,),
                      mode=lax.GatherScatterMode.PROMISE_IN_BOUNDS)


def _wid():
    return lax.axis_index("s") * _i32(NC) + lax.axis_index("c")


def _make_agg(W, BN, NCH, BPAD):
    @functools.partial(
        pl.kernel,
        out_type=jax.ShapeDtypeStruct((NPAD, W), jnp.float32),
        mesh=_MESH,
        scratch_types=[
            pltpu.VMEM((BPAD,), jnp.int32),
            pltpu.VMEM((SE,), jnp.int32),
            pltpu.VMEM((SE,), jnp.int32),
            pltpu.VMEM((SE,), jnp.float32),
            pltpu.VMEM((2, K, W), jnp.float32),
            pltpu.VMEM((BN, W), jnp.float32),
            pltpu.SemaphoreType.DMA,
            pltpu.SemaphoreType.DMA,
        ],
        compiler_params=_SC_PARAMS,
    )
    def agg(hs_hbm, row_hbm, col_hbm, nrm_hbm, bnd_hbm, out_hbm,
            bnd_v, stg_r, stg_c, stg_n, rows2, acc_v, sem0, sem1):
        wid = _wid()
        sems = (sem0, sem1)
        pltpu.sync_copy(bnd_hbm, bnd_v)
        nbmax = SE // K

        def chunk(ci, carry):
            c = wid + ci * _i32(NW)
            n0 = c * _i32(BN)

            def zero(i, cz):
                for f in range(W // LANES):
                    acc_v[i, pl.ds(f * LANES, LANES)] = jnp.zeros(
                        (LANES,), jnp.float32)
                return cz

            lax.fori_loop(_i32(0), _i32(BN), zero, _i32(0))

            bv = bnd_v[pl.ds(c, LANES)]
            e0 = bv[0]
            e1 = bv[1]
            e0a = (e0 // _i32(8)) * _i32(8)
            nstage = (e1 - e0a + _i32(SE - 1)) // _i32(SE)

            def start(b, par):
                pltpu.async_copy(
                    hs_hbm.at[stg_r.at[pl.ds(b * _i32(K), K)]],
                    rows2.at[_i32(par)], sems[par])

            def wait(par):
                pltpu.make_async_copy(
                    hs_hbm.at[pl.ds(_i32(0), K)], rows2.at[_i32(par)],
                    sems[par]).wait()

            def stage(sidx, cs):
                sbase = e0a + sidx * _i32(SE)
                pltpu.sync_copy(row_hbm.at[pl.ds(sbase, SE)], stg_r)
                pltpu.sync_copy(col_hbm.at[pl.ds(sbase, SE)], stg_c)
                pltpu.sync_copy(nrm_hbm.at[pl.ds(sbase, SE)], stg_n)
                nb = jnp.minimum(_i32(nbmax),
                                 (e1 - sbase + _i32(K - 1)) // _i32(K))

                @pl.when(nb > _i32(0))
                def _():
                    start(_i32(0), 0)

                def accum(b, par):
                    cv = stg_c[pl.ds(b * _i32(K), K)] - n0
                    nv = stg_n[pl.ds(b * _i32(K), K)]
                    ev = (jnp.arange(LANES, dtype=jnp.int32) + sbase
                          + b * _i32(K))

                    def edge(j, ce):
                        jv = jnp.full((LANES,), j, jnp.int32)
                        rv = _dyn_gather(cv, jv)
                        wv = _dyn_gather(nv, jv)
                        ejv = _dyn_gather(ev, jv)
                        mv = jnp.logical_and(ejv >= e0, ejv < e1)
                        for f in range(W // LANES):
                            plsc.addupdate_scatter(
                                acc_v,
                                [rv, jnp.arange(LANES, dtype=jnp.int32)
                                 + _i32(f * LANES)],
                                rows2[_i32(par), j, pl.ds(f * LANES, LANES)]
                                * wv,
                                mask=mv)
                        return ce

                    lax.fori_loop(_i32(0), _i32(K), edge, _i32(0))

                def pair(pi, cp):
                    for par in (0, 1):
                        b = _i32(2) * pi + _i32(par)

                        @pl.when(b < nb)
                        def _():
                            @pl.when(b + _i32(1) < nb)
                            def _():
                                start(b + _i32(1), 1 - par)

                            wait(par)
                            accum(b, par)
                    return cp

                lax.fori_loop(_i32(0), (nb + _i32(1)) // _i32(2), pair,
                              _i32(0))
                return cs

            lax.fori_loop(_i32(0), nstage, stage, _i32(0))
            pltpu.sync_copy(acc_v, out_hbm.at[pl.ds(n0, BN)])
            return carry

        lax.fori_loop(_i32(0), _i32(NCH // NW), chunk, _i32(0))

    return agg


_AGG = {1024: _make_agg(1024, 64, 160, 176),
        512: _make_agg(512, 160, 64, 80),
        256: _make_agg(256, 160, 64, 80),
        128: _make_agg(128, 160, 64, 80)}


def _pad_rows(a):
    return jnp.pad(a, ((0, NPAD - a.shape[0]), (0, 0)))


def kernel(x, edge_index, batch, params):
    p = params
    row = edge_index[0].astype(jnp.int32)
    col = edge_index[1].astype(jnp.int32)
    order = jnp.argsort(col, stable=True)
    row_s = row[order]
    col_s = col[order]
    row_p = jnp.pad(row_s, (0, LE - N_EDGES))
    col_p = jnp.pad(col_s, (0, LE - N_EDGES))
    b64 = jnp.searchsorted(
        col_s, (jnp.arange(161) * 64).astype(jnp.int32)).astype(jnp.int32)
    b64 = jnp.pad(b64, (0, 176 - 161))
    b160 = jnp.searchsorted(
        col_s, (jnp.arange(65) * 160).astype(jnp.int32)).astype(jnp.int32)
    b160 = jnp.pad(b160, (0, 80 - 65))
    BNDS = {1024: b64, 512: b160, 256: b160}

    order_r = jnp.argsort(row, stable=True)
    row_rs = row[order_r]
    row_rs_p = jnp.pad(row_rs, (0, LE - N_EDGES))
    rb160 = jnp.searchsorted(
        row_rs, (jnp.arange(65) * 160).astype(jnp.int32)).astype(jnp.int32)
    rb160 = jnp.pad(rb160, (0, 80 - 65))
    ones_h = jnp.ones((NPAD, 128), jnp.float32)
    ones_e = jnp.ones((LE,), jnp.float32)
    deg = 1.0 + _AGG[128](ones_h, row_rs_p, row_rs_p, ones_e, rb160)[:N_NODES, 0]
    dinv = jnp.where(deg > 0, deg ** -0.5, 0.0)

    norm_s = dinv[row_s] * dinv[col_s]
    norm_p = jnp.pad(norm_s, (0, LE - N_EDGES))
    selfw = (dinv * dinv)[:, None]

    def gcn(h, Wm, b, bng, bnb):
        z = h @ Wm
        F = z.shape[1]
        E = _AGG[F](_pad_rows(z), row_p, col_p, norm_p, BNDS[F])[:N_NODES]
        y = jax.nn.relu(E + selfw * z + b)
        m = y.mean(axis=0)
        v = y.var(axis=0)
        return bng * (y - m) / jnp.sqrt(v + EPS) + bnb

    g = gcn(x, p['W1'], p['b1'], p['bn1_g'], p['bn1_b'])
    g = gcn(g, p['W2'], p['b2'], p['bn2_g'], p['bn2_b'])
    g = gcn(g, p['W3'], p['b3'], p['bn3_g'], p['bn3_b'])
    g = gcn(g, p['W4'], p['b4'], p['bn4_g'], p['bn4_b'])
    g5 = gcn(g, p['W5'], p['b5'], p['bn5_g'], p['bn5_b'])

    gate = g5 @ p['gate_W'] + p['gate_b']
    gmax = jax.ops.segment_max(gate, batch, num_segments=N_GRAPHS)
    gmax = jnp.where(jnp.isfinite(gmax), gmax, 0.0)
    e = jnp.exp(gate - gmax[batch])
    denom = jax.ops.segment_sum(e, batch, num_segments=N_GRAPHS)
    d = denom[batch]
    alpha = e / jnp.where(d > 0, d, 1.0)
    h = jax.ops.segment_sum(alpha * g5, batch, num_segments=N_GRAPHS)
    h = jax.nn.relu(h @ p['fc2_W'] + p['fc2_b'])
    h = jax.nn.relu(h @ p['fc3_W'] + p['fc3_b'])
    return h @ p['fc4_W'] + p['fc4_b']
